# baseline (speedup 1.0000x reference)
import functools

import jax
import jax.numpy as jnp
from jax import lax
from jax.experimental import pallas as pl
from jax.experimental.pallas import tpu as pltpu
from jax.experimental.pallas import tpu_sc as plsc

_CHANNELS = 16
_SCALAR_MAX = 4
_BASIS = 9
_OUT_W = 2 * _BASIS * _CHANNELS
_NPAIR = 81

_AN_IDX = jnp.array([0, 0, 0, 0, 0, 0, 1, 2, 3], jnp.int32)
_AN_VALID = jnp.array([False, True, False, False, False, False, True, True, True])
_SDIMS = jnp.array([3, 4, 4, 4], jnp.int32)

_LANES = 16
_B = 3200


def _build_table(w):
    ia = _AN_IDX[:, None]
    ib = _AN_IDX[None, :]
    valid = _AN_VALID[:, None] & _AN_VALID[None, :]
    sfa = w[ia * 4 + ib].reshape(9, 9, _SCALAR_MAX, _CHANNELS)
    sfb = w[ib * 4 + ia].reshape(9, 9, _SCALAR_MAX, _CHANNELS)
    pad = ((0, 0), (0, 0), (0, _BASIS - _SCALAR_MAX), (0, 0))
    sfa_p = jnp.pad(sfa, pad)
    sfb_p = jnp.pad(sfb, pad)
    rows = jnp.arange(_BASIS)[None, None, :, None]
    ma = valid[:, :, None, None] & (rows < _SDIMS[ia][:, :, None, None])
    mb = valid[:, :, None, None] & (rows < _SDIMS[ib][:, :, None, None])
    ta = jnp.where(ma, sfa_p, 0.0)
    tb = jnp.where(mb, sfb_p, 0.0)
    return jnp.concatenate([ta, tb], axis=-1).reshape(_NPAIR, _OUT_W)


def _sc_pair_kernel(num_workers, n_atoms, e_total):
    epw = e_total // num_workers
    nvec = epw // _LANES
    mesh = plsc.VectorSubcoreMesh(core_axis_name="c", subcore_axis_name="s")

    @functools.partial(
        pl.kernel,
        mesh=mesh,
        compiler_params=pltpu.CompilerParams(
            use_tc_tiling_on_sc=False, needs_layout_passes=False),
        out_type=jax.ShapeDtypeStruct((e_total,), jnp.int32),
        scratch_types=[
            pltpu.VMEM((n_atoms,), jnp.int32),
            pltpu.VMEM((epw,), jnp.int32),
            pltpu.VMEM((epw,), jnp.int32),
            pltpu.VMEM((epw,), jnp.int32),
        ],
    )
    def body(an_hbm, eidx_hbm, pair_hbm, an_v, i0_v, i1_v, pair_v):
        wid = lax.axis_index("s") * 2 + lax.axis_index("c")
        ebase = wid * epw
        pltpu.sync_copy(an_hbm, an_v)
        pltpu.sync_copy(eidx_hbm.at[0, pl.ds(ebase, epw)], i0_v)
        pltpu.sync_copy(eidx_hbm.at[1, pl.ds(ebase, epw)], i1_v)

        def pair_at(sl):
            a0 = plsc.load_gather(an_v, [i0_v[sl]])
            a1 = plsc.load_gather(an_v, [i1_v[sl]])
            pair_v[sl] = a0 * 9 + a1

        def pair_body(i, carry):
            pair_at(pl.ds(i * _LANES, _LANES))
            return carry

        lax.fori_loop(0, nvec, pair_body, 0)
        if epw % _LANES:
            pair_at(pl.ds(epw - _LANES, _LANES))
        pltpu.sync_copy(pair_v, pair_hbm.at[pl.ds(ebase, epw)])

    return body


def _tc_expand_body(pair_ref, thi_ref, tlo_ref, out_ref):
    prow = jnp.broadcast_to(pair_ref[0], (_NPAIR, _B))
    lanes = lax.broadcasted_iota(jnp.int32, (_NPAIR, _B), 0)
    oh = (prow == lanes).astype(jnp.bfloat16)
    dn = (((1,), (0,)), ((), ()))
    res = (lax.dot_general(thi_ref[...], oh, dn,
                           preferred_element_type=jnp.float32)
           + lax.dot_general(tlo_ref[...], oh, dn,
                             preferred_element_type=jnp.float32)
           * jnp.float32(1.0 / 512.0))
    out_ref[...] = res.reshape(_BASIS, 2 * _CHANNELS, _B)


def _tc_expand(pair, table_hi, table_lo, e_total):
    nb = e_total // _B
    grid_spec = pl.GridSpec(
        grid=(nb,),
        in_specs=[
            pl.BlockSpec((1, 1, _B), lambda i: (i, 0, 0)),
            pl.BlockSpec((_OUT_W, _NPAIR), lambda i: (0, 0)),
            pl.BlockSpec((_OUT_W, _NPAIR), lambda i: (0, 0)),
        ],
        out_specs=pl.BlockSpec((_BASIS, 2 * _CHANNELS, _B), lambda i: (0, 0, i)),
    )
    return pl.pallas_call(
        _tc_expand_body,
        grid_spec=grid_spec,
        out_shape=jax.ShapeDtypeStruct(
            (_BASIS, 2 * _CHANNELS, e_total), jnp.float32),
    )(pair.reshape(nb, 1, _B), table_hi, table_lo)


def kernel(atomic_numbers, edge_index, embedding_weight):
    n_atoms = atomic_numbers.shape[0]
    e_total = edge_index.shape[1]
    info = plsc.get_sparse_core_info()
    num_workers = info.num_cores * info.num_subcores
    if e_total % (num_workers * 8) or e_total % _B:
        raise ValueError("unsupported edge count")
    table_t = _build_table(embedding_weight).T
    tbits = lax.bitcast_convert_type(table_t, jnp.uint32)
    hi_f32 = lax.bitcast_convert_type(
        tbits & jnp.uint32(0xFFFF0000), jnp.float32)
    table_hi = hi_f32.astype(jnp.bfloat16)
    table_lo = ((table_t - hi_f32) * 512.0).astype(jnp.bfloat16)
    pair = _sc_pair_kernel(num_workers, n_atoms, e_total)(
        atomic_numbers, edge_index)
    out = _tc_expand(pair, table_hi, table_lo, e_total)
    return (jnp.transpose(out, (2, 0, 1)), edge_index)


# speedup vs baseline: 1.0976x; 1.0976x over previous
import functools

import jax
import jax.numpy as jnp
from jax import lax
from jax.experimental import pallas as pl
from jax.experimental.pallas import tpu as pltpu
from jax.experimental.pallas import tpu_sc as plsc

_CHANNELS = 16
_SCALAR_MAX = 4
_BASIS = 9
_OUT_W = 2 * _BASIS * _CHANNELS
_NPAIR = 81

_AN_IDX = jnp.array([0, 0, 0, 0, 0, 0, 1, 2, 3], jnp.int32)
_AN_VALID = jnp.array([False, True, False, False, False, False, True, True, True])
_SDIMS = jnp.array([3, 4, 4, 4], jnp.int32)

_LANES = 16
_B = 6400


def _build_table(w):
    ia = _AN_IDX[:, None]
    ib = _AN_IDX[None, :]
    valid = _AN_VALID[:, None] & _AN_VALID[None, :]
    sfa = w[ia * 4 + ib].reshape(9, 9, _SCALAR_MAX, _CHANNELS)
    sfb = w[ib * 4 + ia].reshape(9, 9, _SCALAR_MAX, _CHANNELS)
    pad = ((0, 0), (0, 0), (0, _BASIS - _SCALAR_MAX), (0, 0))
    sfa_p = jnp.pad(sfa, pad)
    sfb_p = jnp.pad(sfb, pad)
    rows = jnp.arange(_BASIS)[None, None, :, None]
    ma = valid[:, :, None, None] & (rows < _SDIMS[ia][:, :, None, None])
    mb = valid[:, :, None, None] & (rows < _SDIMS[ib][:, :, None, None])
    ta = jnp.where(ma, sfa_p, 0.0)
    tb = jnp.where(mb, sfb_p, 0.0)
    return jnp.concatenate([ta, tb], axis=-1).reshape(_NPAIR, _OUT_W)


def _sc_pair_kernel(num_workers, n_atoms, e_total):
    epw = e_total // num_workers
    nvec = epw // _LANES
    mesh = plsc.VectorSubcoreMesh(core_axis_name="c", subcore_axis_name="s")

    @functools.partial(
        pl.kernel,
        mesh=mesh,
        compiler_params=pltpu.CompilerParams(
            use_tc_tiling_on_sc=False, needs_layout_passes=False),
        out_type=jax.ShapeDtypeStruct((e_total,), jnp.int32),
        scratch_types=[
            pltpu.VMEM((n_atoms,), jnp.int32),
            pltpu.VMEM((epw,), jnp.int32),
            pltpu.VMEM((epw,), jnp.int32),
            pltpu.VMEM((epw,), jnp.int32),
        ],
    )
    def body(an_hbm, eidx_hbm, pair_hbm, an_v, i0_v, i1_v, pair_v):
        wid = lax.axis_index("s") * 2 + lax.axis_index("c")
        ebase = wid * epw
        pltpu.sync_copy(an_hbm, an_v)
        pltpu.sync_copy(eidx_hbm.at[0, pl.ds(ebase, epw)], i0_v)
        pltpu.sync_copy(eidx_hbm.at[1, pl.ds(ebase, epw)], i1_v)

        def pair_at(sl):
            a0 = plsc.load_gather(an_v, [i0_v[sl]])
            a1 = plsc.load_gather(an_v, [i1_v[sl]])
            pair_v[sl] = a0 * 9 + a1

        def pair_body(i, carry):
            pair_at(pl.ds(i * _LANES, _LANES))
            return carry

        lax.fori_loop(0, nvec, pair_body, 0)
        if epw % _LANES:
            pair_at(pl.ds(epw - _LANES, _LANES))
        pltpu.sync_copy(pair_v, pair_hbm.at[pl.ds(ebase, epw)])

    return body


def _tc_expand_body(pair_ref, thi_ref, tlo_ref, out_ref):
    prow = jnp.broadcast_to(pair_ref[0], (_NPAIR, _B))
    lanes = lax.broadcasted_iota(jnp.int32, (_NPAIR, _B), 0)
    oh = (prow == lanes).astype(jnp.bfloat16)
    dn = (((1,), (0,)), ((), ()))
    res = (lax.dot_general(thi_ref[...], oh, dn,
                           preferred_element_type=jnp.float32)
           + lax.dot_general(tlo_ref[...], oh, dn,
                             preferred_element_type=jnp.float32)
           * jnp.float32(1.0 / 512.0))
    out_ref[...] = res.reshape(_BASIS, 2 * _CHANNELS, _B)


def _tc_expand(pair, table_hi, table_lo, e_total):
    nb = e_total // _B
    grid_spec = pl.GridSpec(
        grid=(nb,),
        in_specs=[
            pl.BlockSpec((1, 1, _B), lambda i: (i, 0, 0)),
            pl.BlockSpec((_OUT_W, _NPAIR), lambda i: (0, 0)),
            pl.BlockSpec((_OUT_W, _NPAIR), lambda i: (0, 0)),
        ],
        out_specs=pl.BlockSpec((_BASIS, 2 * _CHANNELS, _B), lambda i: (0, 0, i)),
    )
    return pl.pallas_call(
        _tc_expand_body,
        grid_spec=grid_spec,
        out_shape=jax.ShapeDtypeStruct(
            (_BASIS, 2 * _CHANNELS, e_total), jnp.float32),
    )(pair.reshape(nb, 1, _B), table_hi, table_lo)


def kernel(atomic_numbers, edge_index, embedding_weight):
    n_atoms = atomic_numbers.shape[0]
    e_total = edge_index.shape[1]
    info = plsc.get_sparse_core_info()
    num_workers = info.num_cores * info.num_subcores
    if e_total % (num_workers * 8) or e_total % _B:
        raise ValueError("unsupported edge count")
    table_t = _build_table(embedding_weight).T
    tbits = lax.bitcast_convert_type(table_t, jnp.uint32)
    hi_f32 = lax.bitcast_convert_type(
        tbits & jnp.uint32(0xFFFF0000), jnp.float32)
    table_hi = hi_f32.astype(jnp.bfloat16)
    table_lo = ((table_t - hi_f32) * 512.0).astype(jnp.bfloat16)
    pair = _sc_pair_kernel(num_workers, n_atoms, e_total)(
        atomic_numbers, edge_index)
    out = _tc_expand(pair, table_hi, table_lo, e_total)
    return (jnp.transpose(out, (2, 0, 1)), edge_index)


# speedup vs baseline: 1.1261x; 1.0259x over previous
import functools

import jax
import jax.numpy as jnp
from jax import lax
from jax.experimental import pallas as pl
from jax.experimental.pallas import tpu as pltpu
from jax.experimental.pallas import tpu_sc as plsc

_CHANNELS = 16
_SCALAR_MAX = 4
_BASIS = 9
_OUT_W = 2 * _BASIS * _CHANNELS
_NPAIR = 81

_AN_IDX = jnp.array([0, 0, 0, 0, 0, 0, 1, 2, 3], jnp.int32)
_AN_VALID = jnp.array([False, True, False, False, False, False, True, True, True])
_SDIMS = jnp.array([3, 4, 4, 4], jnp.int32)

_LANES = 16
_B = 16000


def _build_table(w):
    ia = _AN_IDX[:, None]
    ib = _AN_IDX[None, :]
    valid = _AN_VALID[:, None] & _AN_VALID[None, :]
    sfa = w[ia * 4 + ib].reshape(9, 9, _SCALAR_MAX, _CHANNELS)
    sfb = w[ib * 4 + ia].reshape(9, 9, _SCALAR_MAX, _CHANNELS)
    pad = ((0, 0), (0, 0), (0, _BASIS - _SCALAR_MAX), (0, 0))
    sfa_p = jnp.pad(sfa, pad)
    sfb_p = jnp.pad(sfb, pad)
    rows = jnp.arange(_BASIS)[None, None, :, None]
    ma = valid[:, :, None, None] & (rows < _SDIMS[ia][:, :, None, None])
    mb = valid[:, :, None, None] & (rows < _SDIMS[ib][:, :, None, None])
    ta = jnp.where(ma, sfa_p, 0.0)
    tb = jnp.where(mb, sfb_p, 0.0)
    return jnp.concatenate([ta, tb], axis=-1).reshape(_NPAIR, _OUT_W)


def _sc_pair_kernel(num_workers, n_atoms, e_total):
    epw = e_total // num_workers
    nvec = epw // _LANES
    mesh = plsc.VectorSubcoreMesh(core_axis_name="c", subcore_axis_name="s")

    @functools.partial(
        pl.kernel,
        mesh=mesh,
        compiler_params=pltpu.CompilerParams(
            use_tc_tiling_on_sc=False, needs_layout_passes=False),
        out_type=jax.ShapeDtypeStruct((e_total,), jnp.int32),
        scratch_types=[
            pltpu.VMEM((n_atoms,), jnp.int32),
            pltpu.VMEM((epw,), jnp.int32),
            pltpu.VMEM((epw,), jnp.int32),
            pltpu.VMEM((epw,), jnp.int32),
        ],
    )
    def body(an_hbm, eidx_hbm, pair_hbm, an_v, i0_v, i1_v, pair_v):
        wid = lax.axis_index("s") * 2 + lax.axis_index("c")
        ebase = wid * epw
        pltpu.sync_copy(an_hbm, an_v)
        pltpu.sync_copy(eidx_hbm.at[0, pl.ds(ebase, epw)], i0_v)
        pltpu.sync_copy(eidx_hbm.at[1, pl.ds(ebase, epw)], i1_v)

        def pair_at(sl):
            a0 = plsc.load_gather(an_v, [i0_v[sl]])
            a1 = plsc.load_gather(an_v, [i1_v[sl]])
            pair_v[sl] = a0 * 9 + a1

        def pair_body(i, carry):
            pair_at(pl.ds(i * _LANES, _LANES))
            return carry

        lax.fori_loop(0, nvec, pair_body, 0)
        if epw % _LANES:
            pair_at(pl.ds(epw - _LANES, _LANES))
        pltpu.sync_copy(pair_v, pair_hbm.at[pl.ds(ebase, epw)])

    return body


def _tc_expand_body(pair_ref, thi_ref, tlo_ref, out_ref):
    prow = jnp.broadcast_to(pair_ref[0], (_NPAIR, _B))
    lanes = lax.broadcasted_iota(jnp.int32, (_NPAIR, _B), 0)
    oh = (prow == lanes).astype(jnp.bfloat16)
    dn = (((1,), (0,)), ((), ()))
    res = (lax.dot_general(thi_ref[...], oh, dn,
                           preferred_element_type=jnp.float32)
           + lax.dot_general(tlo_ref[...], oh, dn,
                             preferred_element_type=jnp.float32)
           * jnp.float32(1.0 / 512.0))
    out_ref[...] = res.reshape(_BASIS, 2 * _CHANNELS, _B)


def _tc_expand(pair, table_hi, table_lo, e_total):
    nb = e_total // _B
    grid_spec = pl.GridSpec(
        grid=(nb,),
        in_specs=[
            pl.BlockSpec((1, 1, _B), lambda i: (i, 0, 0)),
            pl.BlockSpec((_OUT_W, _NPAIR), lambda i: (0, 0)),
            pl.BlockSpec((_OUT_W, _NPAIR), lambda i: (0, 0)),
        ],
        out_specs=pl.BlockSpec((_BASIS, 2 * _CHANNELS, _B), lambda i: (0, 0, i)),
    )
    return pl.pallas_call(
        _tc_expand_body,
        grid_spec=grid_spec,
        out_shape=jax.ShapeDtypeStruct(
            (_BASIS, 2 * _CHANNELS, e_total), jnp.float32),
    )(pair.reshape(nb, 1, _B), table_hi, table_lo)


def kernel(atomic_numbers, edge_index, embedding_weight):
    n_atoms = atomic_numbers.shape[0]
    e_total = edge_index.shape[1]
    info = plsc.get_sparse_core_info()
    num_workers = info.num_cores * info.num_subcores
    if e_total % (num_workers * 8) or e_total % _B:
        raise ValueError("unsupported edge count")
    table_t = _build_table(embedding_weight).T
    tbits = lax.bitcast_convert_type(table_t, jnp.uint32)
    hi_f32 = lax.bitcast_convert_type(
        tbits & jnp.uint32(0xFFFF0000), jnp.float32)
    table_hi = hi_f32.astype(jnp.bfloat16)
    table_lo = ((table_t - hi_f32) * 512.0).astype(jnp.bfloat16)
    pair = _sc_pair_kernel(num_workers, n_atoms, e_total)(
        atomic_numbers, edge_index)
    out = _tc_expand(pair, table_hi, table_lo, e_total)
    return (jnp.transpose(out, (2, 0, 1)), edge_index)
